# baseline (speedup 1.0000x reference)
import functools

import jax
import jax.numpy as jnp
from jax import lax
from jax.experimental import pallas as pl
from jax.experimental.pallas import tpu as pltpu
from jax.experimental.pallas import tpu_sc as plsc

N = 50000
DEG = 16
D = 100
DP = 128
V = 100000

NC = 2
NS = 16
NW = NC * NS
LANE = 128
SPW = 2
CHUNK = SPW * LANE
NBUF = 3

IT_N = 9
PAD_N = NW * IT_N * CHUNK

BLK = 512
NBH = 33
NB = 3 * NBH
NP = NB * BLK
PAD_E = NP * DEG
EH = NBH * BLK * DEG


def _run_gather(emb_hbm, idx_v, out_hbm, rows, gsem, osem, wid, iters):
    def gather(t, b):
        for j in range(SPW):
            pltpu.async_copy(emb_hbm.at[idx_v.at[t, j]],
                             rows[b].at[pl.ds(j * LANE, LANE)],
                             gsem[b])

    def gather_wait(t, b):
        for j in range(SPW):
            pltpu.make_async_copy(emb_hbm.at[idx_v.at[t, j]],
                                  rows[b].at[pl.ds(j * LANE, LANE)],
                                  gsem[b]).wait()

    def store(t, b):
        row0 = (t * NW + wid) * CHUNK
        pltpu.async_copy(rows[b], out_hbm.at[pl.ds(row0, CHUNK)], osem[b])

    def store_wait(t, b):
        row0 = (t * NW + wid) * CHUNK
        pltpu.make_async_copy(rows[b], out_hbm.at[pl.ds(row0, CHUNK)],
                              osem[b]).wait()

    gather(0, 0)
    gather(1, 1)

    def step(i, carry):
        for b in range(NBUF):
            t = i * NBUF + b
            gather_wait(t, b)
            store(t, b)
            nb = (b + 2) % NBUF

            @pl.when(t == 0)
            def _():
                gather(2, 2)

            @pl.when((t >= 1) & (t + 2 < iters))
            def _():
                store_wait(t - 1, nb)
                gather(t + 2, nb)
        return carry

    lax.fori_loop(0, iters // NBUF, step, 0)
    store_wait(iters - 3, 0)
    store_wait(iters - 2, 1)
    store_wait(iters - 1, 2)


def _sc_body_a(emb_hbm, idxn_hbm, idxe_hbm, outn_hbm, oute_hbm,
               idxn_v, idxe_v, rows0, rows1, rows2,
               gsem0, gsem1, gsem2, osem0, osem1, osem2):
    wid = lax.axis_index("c") * NS + lax.axis_index("s")
    pltpu.sync_copy(idxn_hbm.at[:, wid], idxn_v)
    pltpu.sync_copy(idxe_hbm.at[:, wid], idxe_v)
    rows = (rows0, rows1, rows2)
    gsem = (gsem0, gsem1, gsem2)
    osem = (osem0, osem1, osem2)
    _run_gather(emb_hbm, idxe_v, oute_hbm, rows, gsem, osem, wid, NBH)
    _run_gather(emb_hbm, idxn_v, outn_hbm, rows, gsem, osem, wid, IT_N)


def _sc_body_b(emb_hbm, idxe_hbm, oute_hbm,
               idxe_v, rows0, rows1, rows2,
               gsem0, gsem1, gsem2, osem0, osem1, osem2):
    wid = lax.axis_index("c") * NS + lax.axis_index("s")
    pltpu.sync_copy(idxe_hbm.at[:, wid], idxe_v)
    rows = (rows0, rows1, rows2)
    gsem = (gsem0, gsem1, gsem2)
    osem = (osem0, osem1, osem2)
    _run_gather(emb_hbm, idxe_v, oute_hbm, rows, gsem, osem, wid, NBH)


_SC_SCRATCH_TAIL = [
    pltpu.SemaphoreType.DMA,
    pltpu.SemaphoreType.DMA,
    pltpu.SemaphoreType.DMA,
    pltpu.SemaphoreType.DMA,
    pltpu.SemaphoreType.DMA,
    pltpu.SemaphoreType.DMA,
]


def _mesh():
    return plsc.VectorSubcoreMesh(core_axis_name="c", subcore_axis_name="s",
                                  num_cores=NC, num_subcores=NS)


@functools.cache
def _sc_gather_a():
    return pl.kernel(
        _sc_body_a,
        out_type=[
            jax.ShapeDtypeStruct((PAD_N, DP), jnp.float32),
            jax.ShapeDtypeStruct((EH, DP), jnp.float32),
        ],
        mesh=_mesh(),
        scratch_types=[
            pltpu.VMEM((IT_N, SPW, LANE), jnp.int32),
            pltpu.VMEM((NBH, SPW, LANE), jnp.int32),
            pltpu.VMEM((CHUNK, DP), jnp.float32),
            pltpu.VMEM((CHUNK, DP), jnp.float32),
            pltpu.VMEM((CHUNK, DP), jnp.float32),
        ] + _SC_SCRATCH_TAIL,
        compiler_params=pltpu.CompilerParams(use_tc_tiling_on_sc=True),
    )


@functools.cache
def _sc_gather_b():
    return pl.kernel(
        _sc_body_b,
        out_type=[jax.ShapeDtypeStruct((EH, DP), jnp.float32)],
        mesh=_mesh(),
        scratch_types=[
            pltpu.VMEM((NBH, SPW, LANE), jnp.int32),
            pltpu.VMEM((CHUNK, DP), jnp.float32),
            pltpu.VMEM((CHUNK, DP), jnp.float32),
            pltpu.VMEM((CHUNK, DP), jnp.float32),
        ] + _SC_SCRATCH_TAIL,
        compiler_params=pltpu.CompilerParams(use_tc_tiling_on_sc=True),
    )


def _tc_body(x0_ref, hne_ref, s_ref, wei_ref, w1a_ref, w2x_ref, w2m_ref,
             p_ref, o_ref):
    f32 = jnp.float32
    h3 = hne_ref[...]
    s = s_ref[...]
    wei = wei_ref[...]
    w1last = p_ref[0, :]
    b1 = p_ref[1, :]
    q1 = p_ref[2, :]
    b2 = p_ref[3, :]

    a2 = (h3[..., :D] * s[:, None, :]).reshape(BLK * DEG, D)
    lin = jnp.dot(a2, w1a_ref[...], preferred_element_type=f32)
    pre = (lin.reshape(BLK, DEG, D)
           + wei[:, :, None] * w1last[None, None, :]
           + b1[None, None, :])
    h = jnp.tanh(pre)
    score = jnp.sum(h * q1[None, None, :], axis=2, keepdims=True)
    e = jnp.exp(score)
    ws = jnp.sum(e * h3, axis=1)
    msg = (lax.slice(ws, (0, 0), (BLK, D))
           * (1.0 / lax.slice(ws, (0, D), (BLK, D + 1))))

    msgw = jnp.dot(msg, w2m_ref[...], preferred_element_type=f32) + b2[None, :]
    x = x0_ref[...][:, :D]
    x = jnp.maximum(
        jnp.dot(x, w2x_ref[...], preferred_element_type=f32) + msgw, 0.0)
    x = jnp.maximum(
        jnp.dot(x, w2x_ref[...], preferred_element_type=f32) + msgw, 0.0)
    o_ref[...] = x


def _tc_call(x0g, hne3, s_pad, wei_pad, w1a, w2x, w2m, p, off):
    return pl.pallas_call(
        _tc_body,
        grid=(NBH,),
        in_specs=[
            pl.BlockSpec((BLK, DP), lambda i: (i + off, 0)),
            pl.BlockSpec((BLK, DEG, DP), lambda i: (i, 0, 0)),
            pl.BlockSpec((BLK, D), lambda i: (i + off, 0)),
            pl.BlockSpec((BLK, DEG), lambda i: (i + off, 0)),
            pl.BlockSpec((D, D), lambda i: (0, 0)),
            pl.BlockSpec((D, D), lambda i: (0, 0)),
            pl.BlockSpec((D, D), lambda i: (0, 0)),
            pl.BlockSpec((8, D), lambda i: (0, 0)),
        ],
        out_specs=pl.BlockSpec((BLK, D), lambda i: (i, 0)),
        out_shape=jax.ShapeDtypeStruct((NBH * BLK, D), jnp.float32),
        compiler_params=pltpu.CompilerParams(
            dimension_semantics=("arbitrary",)),
    )(x0g, hne3, s_pad, wei_pad, w1a, w2x, w2m, p)


def kernel(nodes, nei, wei, s_vec, emb, W1_w, W1_b, q1_w, W2_w, W2_b):
    i32 = jnp.int32
    f32 = jnp.float32
    nei_flat = nei.reshape(-1).astype(i32)
    idxn = jnp.concatenate(
        [nodes.astype(i32), jnp.arange(PAD_N - N, dtype=i32)])
    idxn = idxn.reshape(IT_N, NW, SPW, LANE)
    idxe1 = nei_flat[:EH].reshape(NBH, NW, SPW, LANE)
    idxe2 = nei_flat[EH:2 * EH].reshape(NBH, NW, SPW, LANE)
    idxe3 = jnp.concatenate(
        [nei_flat[2 * EH:],
         jnp.arange(PAD_E - N * DEG, dtype=i32)]).reshape(
             NBH, NW, SPW, LANE)
    embp = jnp.pad(emb, ((0, 0), (0, DP - D))).at[:, D].set(1.0)

    x0g, hne1 = _sc_gather_a()(embp, idxn, idxe1)
    hne2, = _sc_gather_b()(embp, idxe2)
    hne3, = _sc_gather_b()(embp, idxe3)

    w1a = W1_w[:, :D].T
    w2x = W2_w[:, :D].T
    w2m = W2_w[:, D:].T
    p = jnp.zeros((8, D), f32)
    p = p.at[0].set(W1_w[:, D])
    p = p.at[1].set(W1_b)
    p = p.at[2].set(q1_w[0])
    p = p.at[3].set(W2_b)

    args = (jnp.pad(s_vec, ((0, NP - N), (0, 0))),
            jnp.pad(wei, ((0, NP - N), (0, 0))),
            w1a, w2x, w2m, p)
    outs = [
        _tc_call(x0g, h.reshape(NBH * BLK, DEG, DP), *args, off=k * NBH)
        for k, h in enumerate((hne1, hne2, hne3))
    ]
    return jnp.concatenate(outs, axis=0)[:N]


# speedup vs baseline: 1.1746x; 1.1746x over previous
import functools

import jax
import jax.numpy as jnp
from jax import lax
from jax.experimental import pallas as pl
from jax.experimental.pallas import tpu as pltpu
from jax.experimental.pallas import tpu_sc as plsc

N = 50000
DEG = 16
D = 100
DP = 128
V = 100000

NC = 2
NS = 16
NW = NC * NS
LANE = 128
SPW = 2
CHUNK = SPW * LANE
NBUF = 3

IT_N = 9
PAD_N = NW * IT_N * CHUNK

BLK = 512
NBH = 33
NB = 3 * NBH
NP = NB * BLK
PAD_E = NP * DEG
EH = NBH * BLK * DEG


def _run_gather(emb_hbm, idx_v, out_hbm, rows, gsem, osem, wid, iters):
    def gather(t, b):
        for j in range(SPW):
            pltpu.async_copy(emb_hbm.at[idx_v.at[t, j]],
                             rows[b].at[pl.ds(j * LANE, LANE)],
                             gsem[b])

    def gather_wait(t, b):
        for j in range(SPW):
            pltpu.make_async_copy(emb_hbm.at[idx_v.at[t, j]],
                                  rows[b].at[pl.ds(j * LANE, LANE)],
                                  gsem[b]).wait()

    def store(t, b):
        row0 = (t * NW + wid) * CHUNK
        pltpu.async_copy(rows[b], out_hbm.at[pl.ds(row0, CHUNK)], osem[b])

    def store_wait(t, b):
        row0 = (t * NW + wid) * CHUNK
        pltpu.make_async_copy(rows[b], out_hbm.at[pl.ds(row0, CHUNK)],
                              osem[b]).wait()

    gather(0, 0)
    gather(1, 1)

    def step(i, carry):
        for b in range(NBUF):
            t = i * NBUF + b
            gather_wait(t, b)
            store(t, b)
            nb = (b + 2) % NBUF

            @pl.when(t == 0)
            def _():
                gather(2, 2)

            @pl.when((t >= 1) & (t + 2 < iters))
            def _():
                store_wait(t - 1, nb)
                gather(t + 2, nb)
        return carry

    lax.fori_loop(0, iters // NBUF, step, 0)
    store_wait(iters - 3, 0)
    store_wait(iters - 2, 1)
    store_wait(iters - 1, 2)


def _sc_body_a(emb_hbm, idx_hbm, outn_hbm, oute_hbm,
               idxn_v, idxe_v, rows0, rows1, rows2,
               gsem0, gsem1, gsem2, osem0, osem1, osem2):
    wid = lax.axis_index("c") * NS + lax.axis_index("s")
    pltpu.sync_copy(idx_hbm.at[pl.ds(0, NBH), wid], idxe_v)
    pltpu.sync_copy(idx_hbm.at[pl.ds(3 * NBH, IT_N), wid], idxn_v)
    rows = (rows0, rows1, rows2)
    gsem = (gsem0, gsem1, gsem2)
    osem = (osem0, osem1, osem2)
    _run_gather(emb_hbm, idxe_v, oute_hbm, rows, gsem, osem, wid, NBH)
    _run_gather(emb_hbm, idxn_v, outn_hbm, rows, gsem, osem, wid, IT_N)


def _make_sc_body_b(base):
    def body(emb_hbm, idx_hbm, oute_hbm,
             idxe_v, rows0, rows1, rows2,
             gsem0, gsem1, gsem2, osem0, osem1, osem2):
        wid = lax.axis_index("c") * NS + lax.axis_index("s")
        pltpu.sync_copy(idx_hbm.at[pl.ds(base, NBH), wid], idxe_v)
        rows = (rows0, rows1, rows2)
        gsem = (gsem0, gsem1, gsem2)
        osem = (osem0, osem1, osem2)
        _run_gather(emb_hbm, idxe_v, oute_hbm, rows, gsem, osem, wid, NBH)
    return body


_SC_SCRATCH_TAIL = [
    pltpu.SemaphoreType.DMA,
    pltpu.SemaphoreType.DMA,
    pltpu.SemaphoreType.DMA,
    pltpu.SemaphoreType.DMA,
    pltpu.SemaphoreType.DMA,
    pltpu.SemaphoreType.DMA,
]


def _mesh():
    return plsc.VectorSubcoreMesh(core_axis_name="c", subcore_axis_name="s",
                                  num_cores=NC, num_subcores=NS)


@functools.cache
def _sc_gather_a():
    return pl.kernel(
        _sc_body_a,
        out_type=[
            jax.ShapeDtypeStruct((PAD_N, DP), jnp.float32),
            jax.ShapeDtypeStruct((EH, DP), jnp.float32),
        ],
        mesh=_mesh(),
        scratch_types=[
            pltpu.VMEM((IT_N, SPW, LANE), jnp.int32),
            pltpu.VMEM((NBH, SPW, LANE), jnp.int32),
            pltpu.VMEM((CHUNK, DP), jnp.float32),
            pltpu.VMEM((CHUNK, DP), jnp.float32),
            pltpu.VMEM((CHUNK, DP), jnp.float32),
        ] + _SC_SCRATCH_TAIL,
        compiler_params=pltpu.CompilerParams(use_tc_tiling_on_sc=True),
    )


@functools.cache
def _sc_gather_b(base):
    return pl.kernel(
        _make_sc_body_b(base),
        out_type=[jax.ShapeDtypeStruct((EH, DP), jnp.float32)],
        mesh=_mesh(),
        scratch_types=[
            pltpu.VMEM((NBH, SPW, LANE), jnp.int32),
            pltpu.VMEM((CHUNK, DP), jnp.float32),
            pltpu.VMEM((CHUNK, DP), jnp.float32),
            pltpu.VMEM((CHUNK, DP), jnp.float32),
        ] + _SC_SCRATCH_TAIL,
        compiler_params=pltpu.CompilerParams(use_tc_tiling_on_sc=True),
    )


def _tc_body(x0_ref, hne_ref, s_ref, wei_ref, w1a_ref, w2x_ref, w2m_ref,
             p_ref, o_ref):
    f32 = jnp.float32
    h3 = hne_ref[...]
    s = s_ref[...]
    wei = wei_ref[...]
    w1last = p_ref[0, :]
    b1 = p_ref[1, :]
    q1 = p_ref[2, :]
    b2 = p_ref[3, :]

    a2 = (h3[..., :D] * s[:, None, :]).reshape(BLK * DEG, D)
    lin = jnp.dot(a2, w1a_ref[...], preferred_element_type=f32)
    pre = (lin.reshape(BLK, DEG, D)
           + wei[:, :, None] * w1last[None, None, :]
           + b1[None, None, :])
    h = jnp.tanh(pre)
    score = jnp.sum(h * q1[None, None, :], axis=2, keepdims=True)
    e = jnp.exp(score)
    ws = jnp.sum(e * h3, axis=1)
    msg = (lax.slice(ws, (0, 0), (BLK, D))
           * (1.0 / lax.slice(ws, (0, D), (BLK, D + 1))))

    msgw = jnp.dot(msg, w2m_ref[...], preferred_element_type=f32) + b2[None, :]
    x = x0_ref[...][:, :D]
    x = jnp.maximum(
        jnp.dot(x, w2x_ref[...], preferred_element_type=f32) + msgw, 0.0)
    x = jnp.maximum(
        jnp.dot(x, w2x_ref[...], preferred_element_type=f32) + msgw, 0.0)
    o_ref[...] = x


def _tc_call(x0g, hne3, s_pad, wei_pad, w1a, w2x, w2m, p, off):
    return pl.pallas_call(
        _tc_body,
        grid=(NBH,),
        in_specs=[
            pl.BlockSpec((BLK, DP), lambda i: (i + off, 0)),
            pl.BlockSpec((BLK, DEG, DP), lambda i: (i, 0, 0)),
            pl.BlockSpec((BLK, D), lambda i: (i + off, 0)),
            pl.BlockSpec((BLK, DEG), lambda i: (i + off, 0)),
            pl.BlockSpec((D, D), lambda i: (0, 0)),
            pl.BlockSpec((D, D), lambda i: (0, 0)),
            pl.BlockSpec((D, D), lambda i: (0, 0)),
            pl.BlockSpec((8, D), lambda i: (0, 0)),
        ],
        out_specs=pl.BlockSpec((BLK, D), lambda i: (i, 0)),
        out_shape=jax.ShapeDtypeStruct((NBH * BLK, D), jnp.float32),
        compiler_params=pltpu.CompilerParams(
            dimension_semantics=("arbitrary",)),
    )(x0g, hne3, s_pad, wei_pad, w1a, w2x, w2m, p)


def kernel(nodes, nei, wei, s_vec, emb, W1_w, W1_b, q1_w, W2_w, W2_b):
    i32 = jnp.int32
    f32 = jnp.float32
    idx_all = jnp.concatenate(
        [nei.reshape(-1).astype(i32),
         jnp.arange(PAD_E - N * DEG, dtype=i32),
         nodes.astype(i32),
         jnp.arange(PAD_N - N, dtype=i32)]).reshape(
             3 * NBH + IT_N, NW, SPW, LANE)
    onescol = jnp.zeros((DP - D,), f32).at[0].set(1.0)
    embp = jnp.concatenate(
        [emb, jnp.broadcast_to(onescol[None, :], (V, DP - D))], axis=1)

    x0g, hne1 = _sc_gather_a()(embp, idx_all)
    hne2, = _sc_gather_b(NBH)(embp, idx_all)
    hne3, = _sc_gather_b(2 * NBH)(embp, idx_all)

    w1a = W1_w[:, :D].T
    w2x = W2_w[:, :D].T
    w2m = W2_w[:, D:].T
    p = jnp.zeros((8, D), f32)
    p = p.at[0].set(W1_w[:, D])
    p = p.at[1].set(W1_b)
    p = p.at[2].set(q1_w[0])
    p = p.at[3].set(W2_b)

    args = (jnp.pad(s_vec, ((0, NP - N), (0, 0))),
            jnp.pad(wei, ((0, NP - N), (0, 0))),
            w1a, w2x, w2m, p)
    outs = [
        _tc_call(x0g, h.reshape(NBH * BLK, DEG, DP), *args, off=k * NBH)
        for k, h in enumerate((hne1, hne2, hne3))
    ]
    return jnp.concatenate(outs, axis=0)[:N]


# speedup vs baseline: 1.1772x; 1.0022x over previous
import functools

import jax
import jax.numpy as jnp
from jax import lax
from jax.experimental import pallas as pl
from jax.experimental.pallas import tpu as pltpu
from jax.experimental.pallas import tpu_sc as plsc

N = 50000
DEG = 16
D = 100
DP = 128
V = 100000

NC = 2
NS = 16
NW = NC * NS
LANE = 128
SPW = 2
CHUNK = SPW * LANE
NBUF = 3

IT_N = 9
PAD_N = NW * IT_N * CHUNK

BLK = 512
STAGES = (6, 30, 30, 33)
OFFS = (0, 6, 36, 66)
NB = sum(STAGES)
NP = NB * BLK
PAD_E = NP * DEG


def _run_gather(emb_hbm, idx_v, out_hbm, rows, gsem, osem, wid, iters):
    def gather(t, b):
        for j in range(SPW):
            pltpu.async_copy(emb_hbm.at[idx_v.at[t, j]],
                             rows[b].at[pl.ds(j * LANE, LANE)],
                             gsem[b])

    def gather_wait(t, b):
        for j in range(SPW):
            pltpu.make_async_copy(emb_hbm.at[idx_v.at[t, j]],
                                  rows[b].at[pl.ds(j * LANE, LANE)],
                                  gsem[b]).wait()

    def store(t, b):
        row0 = (t * NW + wid) * CHUNK
        pltpu.async_copy(rows[b], out_hbm.at[pl.ds(row0, CHUNK)], osem[b])

    def store_wait(t, b):
        row0 = (t * NW + wid) * CHUNK
        pltpu.make_async_copy(rows[b], out_hbm.at[pl.ds(row0, CHUNK)],
                              osem[b]).wait()

    gather(0, 0)
    gather(1, 1)

    def step(i, carry):
        for b in range(NBUF):
            t = i * NBUF + b
            gather_wait(t, b)
            store(t, b)
            nb = (b + 2) % NBUF

            @pl.when(t == 0)
            def _():
                gather(2, 2)

            @pl.when((t >= 1) & (t + 2 < iters))
            def _():
                store_wait(t - 1, nb)
                gather(t + 2, nb)
        return carry

    lax.fori_loop(0, iters // NBUF, step, 0)
    store_wait(iters - 3, 0)
    store_wait(iters - 2, 1)
    store_wait(iters - 1, 2)


def _sc_body_a(emb_hbm, idx_hbm, outn_hbm, oute_hbm,
               idxn_v, idxe_v, rows0, rows1, rows2,
               gsem0, gsem1, gsem2, osem0, osem1, osem2):
    wid = lax.axis_index("c") * NS + lax.axis_index("s")
    pltpu.sync_copy(idx_hbm.at[pl.ds(0, STAGES[0]), wid], idxe_v)
    pltpu.sync_copy(idx_hbm.at[pl.ds(NB, IT_N), wid], idxn_v)
    rows = (rows0, rows1, rows2)
    gsem = (gsem0, gsem1, gsem2)
    osem = (osem0, osem1, osem2)
    _run_gather(emb_hbm, idxe_v, oute_hbm, rows, gsem, osem, wid, STAGES[0])
    _run_gather(emb_hbm, idxn_v, outn_hbm, rows, gsem, osem, wid, IT_N)


def _make_sc_body_b(base, nb):
    def body(emb_hbm, idx_hbm, oute_hbm,
             idxe_v, rows0, rows1, rows2,
             gsem0, gsem1, gsem2, osem0, osem1, osem2):
        wid = lax.axis_index("c") * NS + lax.axis_index("s")
        pltpu.sync_copy(idx_hbm.at[pl.ds(base, nb), wid], idxe_v)
        rows = (rows0, rows1, rows2)
        gsem = (gsem0, gsem1, gsem2)
        osem = (osem0, osem1, osem2)
        _run_gather(emb_hbm, idxe_v, oute_hbm, rows, gsem, osem, wid, nb)
    return body


_SC_SCRATCH_TAIL = [
    pltpu.SemaphoreType.DMA,
    pltpu.SemaphoreType.DMA,
    pltpu.SemaphoreType.DMA,
    pltpu.SemaphoreType.DMA,
    pltpu.SemaphoreType.DMA,
    pltpu.SemaphoreType.DMA,
]


def _mesh():
    return plsc.VectorSubcoreMesh(core_axis_name="c", subcore_axis_name="s",
                                  num_cores=NC, num_subcores=NS)


@functools.cache
def _sc_gather_a():
    return pl.kernel(
        _sc_body_a,
        out_type=[
            jax.ShapeDtypeStruct((PAD_N, DP), jnp.float32),
            jax.ShapeDtypeStruct((STAGES[0] * BLK * DEG, DP), jnp.float32),
        ],
        mesh=_mesh(),
        scratch_types=[
            pltpu.VMEM((IT_N, SPW, LANE), jnp.int32),
            pltpu.VMEM((STAGES[0], SPW, LANE), jnp.int32),
            pltpu.VMEM((CHUNK, DP), jnp.float32),
            pltpu.VMEM((CHUNK, DP), jnp.float32),
            pltpu.VMEM((CHUNK, DP), jnp.float32),
        ] + _SC_SCRATCH_TAIL,
        compiler_params=pltpu.CompilerParams(use_tc_tiling_on_sc=True),
    )


@functools.cache
def _sc_gather_b(base, nb):
    return pl.kernel(
        _make_sc_body_b(base, nb),
        out_type=[jax.ShapeDtypeStruct((nb * BLK * DEG, DP), jnp.float32)],
        mesh=_mesh(),
        scratch_types=[
            pltpu.VMEM((nb, SPW, LANE), jnp.int32),
            pltpu.VMEM((CHUNK, DP), jnp.float32),
            pltpu.VMEM((CHUNK, DP), jnp.float32),
            pltpu.VMEM((CHUNK, DP), jnp.float32),
        ] + _SC_SCRATCH_TAIL,
        compiler_params=pltpu.CompilerParams(use_tc_tiling_on_sc=True),
    )


def _tc_body(x0_ref, hne_ref, s_ref, wei_ref, w1a_ref, w2x_ref, w2m_ref,
             p_ref, o_ref):
    f32 = jnp.float32
    h3 = hne_ref[...]
    s = s_ref[...]
    wei = wei_ref[...]
    w1last = p_ref[0, :]
    b1 = p_ref[1, :]
    q1 = p_ref[2, :]
    b2 = p_ref[3, :]

    a2 = (h3[..., :D] * s[:, None, :]).reshape(BLK * DEG, D)
    lin = jnp.dot(a2, w1a_ref[...], preferred_element_type=f32)
    pre = (lin.reshape(BLK, DEG, D)
           + wei[:, :, None] * w1last[None, None, :]
           + b1[None, None, :])
    h = jnp.tanh(pre)
    score = jnp.sum(h * q1[None, None, :], axis=2, keepdims=True)
    e = jnp.exp(score)
    ws = jnp.sum(e * h3, axis=1)
    msg = (lax.slice(ws, (0, 0), (BLK, D))
           * (1.0 / lax.slice(ws, (0, D), (BLK, D + 1))))

    msgw = jnp.dot(msg, w2m_ref[...], preferred_element_type=f32) + b2[None, :]
    x = x0_ref[...][:, :D]
    x = jnp.maximum(
        jnp.dot(x, w2x_ref[...], preferred_element_type=f32) + msgw, 0.0)
    x = jnp.maximum(
        jnp.dot(x, w2x_ref[...], preferred_element_type=f32) + msgw, 0.0)
    o_ref[...] = x


def _tc_call(x0g, hne3, s_pad, wei_pad, w1a, w2x, w2m, p, nb, off):
    return pl.pallas_call(
        _tc_body,
        grid=(nb,),
        in_specs=[
            pl.BlockSpec((BLK, DP), lambda i: (i + off, 0)),
            pl.BlockSpec((BLK, DEG, DP), lambda i: (i, 0, 0)),
            pl.BlockSpec((BLK, D), lambda i: (i + off, 0)),
            pl.BlockSpec((BLK, DEG), lambda i: (i + off, 0)),
            pl.BlockSpec((D, D), lambda i: (0, 0)),
            pl.BlockSpec((D, D), lambda i: (0, 0)),
            pl.BlockSpec((D, D), lambda i: (0, 0)),
            pl.BlockSpec((8, D), lambda i: (0, 0)),
        ],
        out_specs=pl.BlockSpec((BLK, D), lambda i: (i, 0)),
        out_shape=jax.ShapeDtypeStruct((nb * BLK, D), jnp.float32),
        compiler_params=pltpu.CompilerParams(
            dimension_semantics=("arbitrary",)),
    )(x0g, hne3, s_pad, wei_pad, w1a, w2x, w2m, p)


def kernel(nodes, nei, wei, s_vec, emb, W1_w, W1_b, q1_w, W2_w, W2_b):
    i32 = jnp.int32
    f32 = jnp.float32
    idx_all = jnp.concatenate(
        [nei.reshape(-1).astype(i32),
         jnp.arange(PAD_E - N * DEG, dtype=i32),
         nodes.astype(i32),
         jnp.arange(PAD_N - N, dtype=i32)]).reshape(
             NB + IT_N, NW, SPW, LANE)
    onescol = jnp.zeros((DP - D,), f32).at[0].set(1.0)
    embp = jnp.concatenate(
        [emb, jnp.broadcast_to(onescol[None, :], (V, DP - D))], axis=1)

    x0g, hne0 = _sc_gather_a()(embp, idx_all)
    hnes = [hne0]
    for k in range(1, len(STAGES)):
        h, = _sc_gather_b(OFFS[k], STAGES[k])(embp, idx_all)
        hnes.append(h)

    w1a = W1_w[:, :D].T
    w2x = W2_w[:, :D].T
    w2m = W2_w[:, D:].T
    p = jnp.zeros((8, D), f32)
    p = p.at[0].set(W1_w[:, D])
    p = p.at[1].set(W1_b)
    p = p.at[2].set(q1_w[0])
    p = p.at[3].set(W2_b)

    args = (jnp.pad(s_vec, ((0, NP - N), (0, 0))),
            jnp.pad(wei, ((0, NP - N), (0, 0))),
            w1a, w2x, w2m, p)
    outs = [
        _tc_call(x0g, h.reshape(STAGES[k] * BLK, DEG, DP), *args,
                 nb=STAGES[k], off=OFFS[k])
        for k, h in enumerate(hnes)
    ]
    return jnp.concatenate(outs, axis=0)[:N]
